# baseline (speedup 1.0000x reference)
import functools

import jax
import jax.numpy as jnp
from jax import lax
from jax.experimental import pallas as pl
from jax.experimental.pallas import tpu as pltpu
from jax.experimental.pallas import tpu_sc as plsc

B = 200000
F_DIM = 128
E_DIM = 64
OUT_DIM = F_DIM + E_DIM

NW = 32
CHUNK = 320
NCHUNKS = B // CHUNK
CPW = -(-NCHUNKS // NW)
GSUB = 128

TBS = 512
TGRID = -(-B // TBS)


def _sc_gather(ids, emb128):
    mesh = plsc.VectorSubcoreMesh(core_axis_name="c", subcore_axis_name="s")

    @functools.partial(
        pl.kernel,
        mesh=mesh,
        out_type=jax.ShapeDtypeStruct((B, F_DIM), jnp.float32),
        scratch_types=[
            pltpu.VMEM((CHUNK,), jnp.int32),
            pltpu.VMEM((CHUNK, F_DIM), jnp.float32),
            pltpu.SemaphoreType.DMA,
        ],
    )
    def k(ids_hbm, emb_hbm, wide_hbm, idx_v, rows_v, sem_g):
        wid = lax.axis_index("s") * 2 + lax.axis_index("c")

        def step(i, _):
            ci = wid + i * NW

            @pl.when(ci < NCHUNKS)
            def _():
                base = ci * CHUNK
                pltpu.sync_copy(ids_hbm.at[pl.ds(base, CHUNK)], idx_v)
                gathers = []
                for s in range(0, CHUNK, GSUB):
                    n = min(GSUB, CHUNK - s)
                    gathers.append(pltpu.async_copy(
                        emb_hbm.at[idx_v.at[pl.ds(s, n)]],
                        rows_v.at[pl.ds(s, n)], sem_g))
                for g in gathers:
                    g.wait()
                w_wide = pltpu.async_copy(
                    rows_v, wide_hbm.at[pl.ds(base, CHUNK), :], sem_g)
                w_wide.wait()
            return ()

        lax.fori_loop(0, CPW, step, ())

    return k(ids, emb128)


def _tc_feats_t(feats):
    def body(feats_ref, out_ref):
        out_ref[...] = feats_ref[...].T

    return pl.pallas_call(
        body,
        grid=(TGRID,),
        in_specs=[pl.BlockSpec((TBS, F_DIM), lambda i: (i, 0))],
        out_specs=pl.BlockSpec((F_DIM, TBS), lambda i: (0, i)),
        out_shape=jax.ShapeDtypeStruct((OUT_DIM, B), jnp.float32),
    )(feats)


def _tc_emb_t(out_t, wide):
    def body(_, wide_ref, out_ref):
        out_ref[...] = wide_ref[:, 0:E_DIM].T

    return pl.pallas_call(
        body,
        grid=(TGRID,),
        in_specs=[
            pl.BlockSpec(memory_space=pl.ANY),
            pl.BlockSpec((TBS, F_DIM), lambda i: (i, 0)),
        ],
        out_specs=pl.BlockSpec((E_DIM, TBS), lambda i: (2, i)),
        out_shape=jax.ShapeDtypeStruct((OUT_DIM, B), jnp.float32),
        input_output_aliases={0: 0},
    )(out_t, wide)


def kernel(ids, feats, hop_idx, emb_W):
    n_nodes = emb_W.shape[0] - 1
    gather_ids = jnp.where(hop_idx > 0, ids,
                           jnp.full_like(ids, n_nodes)).astype(jnp.int32)
    emb128 = jnp.pad(emb_W, ((0, 0), (0, F_DIM - E_DIM)))
    wide = _sc_gather(gather_ids, emb128)
    out_t = _tc_feats_t(feats)
    out_t = _tc_emb_t(out_t, wide)
    return out_t.T


# speedup vs baseline: 1.9109x; 1.9109x over previous
import functools

import jax
import jax.numpy as jnp
from jax import lax
from jax.experimental import pallas as pl
from jax.experimental.pallas import tpu as pltpu
from jax.experimental.pallas import tpu_sc as plsc

B = 200000
F_DIM = 128
E_DIM = 64
OUT_DIM = F_DIM + E_DIM

NW = 32
CHUNK = 320
NCHUNKS = B // CHUNK
CPW = -(-NCHUNKS // NW)
GSUB = 128

TBS = 2048
TGRID = -(-B // TBS)


def _sc_gather(ids, emb128):
    mesh = plsc.VectorSubcoreMesh(core_axis_name="c", subcore_axis_name="s")

    @functools.partial(
        pl.kernel,
        mesh=mesh,
        out_type=jax.ShapeDtypeStruct((B, F_DIM), jnp.float32),
        scratch_types=[
            pltpu.VMEM((CHUNK,), jnp.int32),
            pltpu.VMEM((CHUNK, F_DIM), jnp.float32),
            pltpu.SemaphoreType.DMA,
        ],
    )
    def k(ids_hbm, emb_hbm, wide_hbm, idx_v, rows_v, sem_g):
        wid = lax.axis_index("s") * 2 + lax.axis_index("c")

        def step(i, _):
            ci = wid + i * NW

            @pl.when(ci < NCHUNKS)
            def _():
                base = ci * CHUNK
                pltpu.sync_copy(ids_hbm.at[pl.ds(base, CHUNK)], idx_v)
                gathers = []
                for s in range(0, CHUNK, GSUB):
                    n = min(GSUB, CHUNK - s)
                    gathers.append(pltpu.async_copy(
                        emb_hbm.at[idx_v.at[pl.ds(s, n)]],
                        rows_v.at[pl.ds(s, n)], sem_g))
                for g in gathers:
                    g.wait()
                w_wide = pltpu.async_copy(
                    rows_v, wide_hbm.at[pl.ds(base, CHUNK), :], sem_g)
                w_wide.wait()
            return ()

        lax.fori_loop(0, CPW, step, ())

    return k(ids, emb128)


def _tc_feats_t(feats):
    def body(feats_ref, out_ref):
        out_ref[...] = feats_ref[...].T

    return pl.pallas_call(
        body,
        grid=(TGRID,),
        in_specs=[pl.BlockSpec((TBS, F_DIM), lambda i: (i, 0))],
        out_specs=pl.BlockSpec((F_DIM, TBS), lambda i: (0, i)),
        out_shape=jax.ShapeDtypeStruct((OUT_DIM, B), jnp.float32),
    )(feats)


def _tc_emb_t(out_t, wide):
    def body(_, wide_ref, out_ref):
        out_ref[...] = wide_ref[:, 0:E_DIM].T

    return pl.pallas_call(
        body,
        grid=(TGRID,),
        in_specs=[
            pl.BlockSpec(memory_space=pl.ANY),
            pl.BlockSpec((TBS, F_DIM), lambda i: (i, 0)),
        ],
        out_specs=pl.BlockSpec((E_DIM, TBS), lambda i: (2, i)),
        out_shape=jax.ShapeDtypeStruct((OUT_DIM, B), jnp.float32),
        input_output_aliases={0: 0},
    )(out_t, wide)


def kernel(ids, feats, hop_idx, emb_W):
    n_nodes = emb_W.shape[0] - 1
    gather_ids = jnp.where(hop_idx > 0, ids,
                           jnp.full_like(ids, n_nodes)).astype(jnp.int32)
    emb128 = jnp.pad(emb_W, ((0, 0), (0, F_DIM - E_DIM)))
    wide = _sc_gather(gather_ids, emb128)
    out_t = _tc_feats_t(feats)
    out_t = _tc_emb_t(out_t, wide)
    return out_t.T


# speedup vs baseline: 2.1403x; 1.1201x over previous
import functools

import jax
import jax.numpy as jnp
from jax import lax
from jax.experimental import pallas as pl
from jax.experimental.pallas import tpu as pltpu
from jax.experimental.pallas import tpu_sc as plsc

B = 200000
F_DIM = 128
E_DIM = 64
OUT_DIM = F_DIM + E_DIM

NW = 32
CHUNK = 320
NCHUNKS = B // CHUNK
CPW = -(-NCHUNKS // NW)
GSUB = 128

TBS = 4096
TGRID = -(-B // TBS)


def _sc_gather(ids, emb128):
    mesh = plsc.VectorSubcoreMesh(core_axis_name="c", subcore_axis_name="s")

    @functools.partial(
        pl.kernel,
        mesh=mesh,
        out_type=jax.ShapeDtypeStruct((B, F_DIM), jnp.float32),
        scratch_types=[
            pltpu.VMEM((CHUNK,), jnp.int32),
            pltpu.VMEM((CHUNK, F_DIM), jnp.float32),
            pltpu.SemaphoreType.DMA,
        ],
    )
    def k(ids_hbm, emb_hbm, wide_hbm, idx_v, rows_v, sem_g):
        wid = lax.axis_index("s") * 2 + lax.axis_index("c")

        def step(i, _):
            ci = wid + i * NW

            @pl.when(ci < NCHUNKS)
            def _():
                base = ci * CHUNK
                pltpu.sync_copy(ids_hbm.at[pl.ds(base, CHUNK)], idx_v)
                gathers = []
                for s in range(0, CHUNK, GSUB):
                    n = min(GSUB, CHUNK - s)
                    gathers.append(pltpu.async_copy(
                        emb_hbm.at[idx_v.at[pl.ds(s, n)]],
                        rows_v.at[pl.ds(s, n)], sem_g))
                for g in gathers:
                    g.wait()
                w_wide = pltpu.async_copy(
                    rows_v, wide_hbm.at[pl.ds(base, CHUNK), :], sem_g)
                w_wide.wait()
            return ()

        lax.fori_loop(0, CPW, step, ())

    return k(ids, emb128)


def _tc_feats_t(feats):
    def body(feats_ref, out_ref):
        out_ref[...] = feats_ref[...].T

    return pl.pallas_call(
        body,
        grid=(TGRID,),
        in_specs=[pl.BlockSpec((TBS, F_DIM), lambda i: (i, 0))],
        out_specs=pl.BlockSpec((F_DIM, TBS), lambda i: (0, i)),
        out_shape=jax.ShapeDtypeStruct((OUT_DIM, B), jnp.float32),
    )(feats)


def _tc_emb_t(out_t, wide):
    def body(_, wide_ref, out_ref):
        out_ref[...] = wide_ref[:, 0:E_DIM].T

    return pl.pallas_call(
        body,
        grid=(TGRID,),
        in_specs=[
            pl.BlockSpec(memory_space=pl.ANY),
            pl.BlockSpec((TBS, F_DIM), lambda i: (i, 0)),
        ],
        out_specs=pl.BlockSpec((E_DIM, TBS), lambda i: (2, i)),
        out_shape=jax.ShapeDtypeStruct((OUT_DIM, B), jnp.float32),
        input_output_aliases={0: 0},
    )(out_t, wide)


def kernel(ids, feats, hop_idx, emb_W):
    n_nodes = emb_W.shape[0] - 1
    gather_ids = jnp.where(hop_idx > 0, ids,
                           jnp.full_like(ids, n_nodes)).astype(jnp.int32)
    emb128 = jnp.pad(emb_W, ((0, 0), (0, F_DIM - E_DIM)))
    wide = _sc_gather(gather_ids, emb128)
    out_t = _tc_feats_t(feats)
    out_t = _tc_emb_t(out_t, wide)
    return out_t.T


# speedup vs baseline: 2.2403x; 1.0467x over previous
import functools

import jax
import jax.numpy as jnp
from jax import lax
from jax.experimental import pallas as pl
from jax.experimental.pallas import tpu as pltpu
from jax.experimental.pallas import tpu_sc as plsc

B = 200000
F_DIM = 128
E_DIM = 64
OUT_DIM = F_DIM + E_DIM

NW = 32
CHUNK = 320
NCHUNKS = B // CHUNK
CPW = -(-NCHUNKS // NW)
GSUB = 128

TBS = 8192
TGRID = -(-B // TBS)


def _sc_gather(ids, emb128):
    mesh = plsc.VectorSubcoreMesh(core_axis_name="c", subcore_axis_name="s")

    @functools.partial(
        pl.kernel,
        mesh=mesh,
        out_type=jax.ShapeDtypeStruct((B, F_DIM), jnp.float32),
        scratch_types=[
            pltpu.VMEM((CHUNK,), jnp.int32),
            pltpu.VMEM((CHUNK, F_DIM), jnp.float32),
            pltpu.SemaphoreType.DMA,
        ],
    )
    def k(ids_hbm, emb_hbm, wide_hbm, idx_v, rows_v, sem_g):
        wid = lax.axis_index("s") * 2 + lax.axis_index("c")

        def step(i, _):
            ci = wid + i * NW

            @pl.when(ci < NCHUNKS)
            def _():
                base = ci * CHUNK
                pltpu.sync_copy(ids_hbm.at[pl.ds(base, CHUNK)], idx_v)
                gathers = []
                for s in range(0, CHUNK, GSUB):
                    n = min(GSUB, CHUNK - s)
                    gathers.append(pltpu.async_copy(
                        emb_hbm.at[idx_v.at[pl.ds(s, n)]],
                        rows_v.at[pl.ds(s, n)], sem_g))
                for g in gathers:
                    g.wait()
                w_wide = pltpu.async_copy(
                    rows_v, wide_hbm.at[pl.ds(base, CHUNK), :], sem_g)
                w_wide.wait()
            return ()

        lax.fori_loop(0, CPW, step, ())

    return k(ids, emb128)


def _tc_feats_t(feats):
    def body(feats_ref, out_ref):
        out_ref[...] = feats_ref[...].T

    return pl.pallas_call(
        body,
        grid=(TGRID,),
        in_specs=[pl.BlockSpec((TBS, F_DIM), lambda i: (i, 0))],
        out_specs=pl.BlockSpec((F_DIM, TBS), lambda i: (0, i)),
        out_shape=jax.ShapeDtypeStruct((OUT_DIM, B), jnp.float32),
    )(feats)


def _tc_emb_t(out_t, wide):
    def body(_, wide_ref, out_ref):
        out_ref[...] = wide_ref[:, 0:E_DIM].T

    return pl.pallas_call(
        body,
        grid=(TGRID,),
        in_specs=[
            pl.BlockSpec(memory_space=pl.ANY),
            pl.BlockSpec((TBS, F_DIM), lambda i: (i, 0)),
        ],
        out_specs=pl.BlockSpec((E_DIM, TBS), lambda i: (2, i)),
        out_shape=jax.ShapeDtypeStruct((OUT_DIM, B), jnp.float32),
        input_output_aliases={0: 0},
    )(out_t, wide)


def kernel(ids, feats, hop_idx, emb_W):
    n_nodes = emb_W.shape[0] - 1
    gather_ids = jnp.where(hop_idx > 0, ids,
                           jnp.full_like(ids, n_nodes)).astype(jnp.int32)
    emb128 = jnp.pad(emb_W, ((0, 0), (0, F_DIM - E_DIM)))
    wide = _sc_gather(gather_ids, emb128)
    out_t = _tc_feats_t(feats)
    out_t = _tc_emb_t(out_t, wide)
    return out_t.T


# speedup vs baseline: 2.2578x; 1.0078x over previous
import functools

import jax
import jax.numpy as jnp
from jax import lax
from jax.experimental import pallas as pl
from jax.experimental.pallas import tpu as pltpu
from jax.experimental.pallas import tpu_sc as plsc

B = 200000
F_DIM = 128
E_DIM = 64
OUT_DIM = F_DIM + E_DIM

NW = 32
CHUNK = 320
NCHUNKS = B // CHUNK
CPW = -(-NCHUNKS // NW)
GSUB = 128

TBS = 16384
TGRID = -(-B // TBS)


def _sc_gather(ids, emb128):
    mesh = plsc.VectorSubcoreMesh(core_axis_name="c", subcore_axis_name="s")

    @functools.partial(
        pl.kernel,
        mesh=mesh,
        out_type=jax.ShapeDtypeStruct((B, F_DIM), jnp.float32),
        scratch_types=[
            pltpu.VMEM((CHUNK,), jnp.int32),
            pltpu.VMEM((CHUNK, F_DIM), jnp.float32),
            pltpu.SemaphoreType.DMA,
        ],
    )
    def k(ids_hbm, emb_hbm, wide_hbm, idx_v, rows_v, sem_g):
        wid = lax.axis_index("s") * 2 + lax.axis_index("c")

        def step(i, _):
            ci = wid + i * NW

            @pl.when(ci < NCHUNKS)
            def _():
                base = ci * CHUNK
                pltpu.sync_copy(ids_hbm.at[pl.ds(base, CHUNK)], idx_v)
                gathers = []
                for s in range(0, CHUNK, GSUB):
                    n = min(GSUB, CHUNK - s)
                    gathers.append(pltpu.async_copy(
                        emb_hbm.at[idx_v.at[pl.ds(s, n)]],
                        rows_v.at[pl.ds(s, n)], sem_g))
                for g in gathers:
                    g.wait()
                w_wide = pltpu.async_copy(
                    rows_v, wide_hbm.at[pl.ds(base, CHUNK), :], sem_g)
                w_wide.wait()
            return ()

        lax.fori_loop(0, CPW, step, ())

    return k(ids, emb128)


def _tc_feats_t(feats):
    def body(feats_ref, out_ref):
        out_ref[...] = feats_ref[...].T

    return pl.pallas_call(
        body,
        grid=(TGRID,),
        in_specs=[pl.BlockSpec((TBS, F_DIM), lambda i: (i, 0))],
        out_specs=pl.BlockSpec((F_DIM, TBS), lambda i: (0, i)),
        out_shape=jax.ShapeDtypeStruct((OUT_DIM, B), jnp.float32),
    )(feats)


def _tc_emb_t(out_t, wide):
    def body(_, wide_ref, out_ref):
        out_ref[...] = wide_ref[:, 0:E_DIM].T

    return pl.pallas_call(
        body,
        grid=(TGRID,),
        in_specs=[
            pl.BlockSpec(memory_space=pl.ANY),
            pl.BlockSpec((TBS, F_DIM), lambda i: (i, 0)),
        ],
        out_specs=pl.BlockSpec((E_DIM, TBS), lambda i: (2, i)),
        out_shape=jax.ShapeDtypeStruct((OUT_DIM, B), jnp.float32),
        input_output_aliases={0: 0},
    )(out_t, wide)


def kernel(ids, feats, hop_idx, emb_W):
    n_nodes = emb_W.shape[0] - 1
    gather_ids = jnp.where(hop_idx > 0, ids,
                           jnp.full_like(ids, n_nodes)).astype(jnp.int32)
    emb128 = jnp.pad(emb_W, ((0, 0), (0, F_DIM - E_DIM)))
    wide = _sc_gather(gather_ids, emb128)
    out_t = _tc_feats_t(feats)
    out_t = _tc_emb_t(out_t, wide)
    return out_t.T
